# baseline (speedup 1.0000x reference)
import functools

import jax
import jax.numpy as jnp
from jax.experimental import pallas as pl
from jax.experimental.pallas import tpu as pltpu

_B_NEIGH = 9
_K_AUG = 256
_MB_BLK1 = 4096
_Q_CHUNK = 448
_MB_BLK2 = 4096
_NCAND = 4
_MB_SHIFT = 256.0
_FV_SHIFT = 128.0


def _mb_aug_kernel(mb_ref, outf_ref, outb_ref):
    mbb = mb_ref[...]
    c = mbb.shape[1]
    mbn = jnp.sum(mbb * mbb, axis=1, keepdims=True)
    one = jnp.ones((mbb.shape[0], 1), jnp.float32)
    z = jnp.zeros((mbb.shape[0], _K_AUG - c - 2), jnp.float32)
    outf_ref[...] = jnp.concatenate([mbb * -2.0, mbn, one, z], axis=1)
    outb_ref[...] = jnp.concatenate(
        [mbb * -2.0, mbn - _MB_SHIFT, one, 288.0 * one, z[:, :-1]],
        axis=1).astype(jnp.bfloat16)


def _fv_aug_kernel(fv_ref, outf_ref, outb_ref):
    fvv = fv_ref[...]
    c = fvv.shape[1]
    qn = jnp.sum(fvv * fvv, axis=1, keepdims=True)
    one = jnp.ones((fvv.shape[0], 1), jnp.float32)
    z = jnp.zeros((fvv.shape[0], _K_AUG - c - 2), jnp.float32)
    outf_ref[...] = jnp.concatenate([fvv, one, qn, z], axis=1)
    outb_ref[...] = jnp.concatenate(
        [fvv, one, qn - _FV_SHIFT, one, z[:, :-1]], axis=1).astype(jnp.bfloat16)


def _pixel_min_kernel(fva_hbm, mba_ref, out_ref, fva_ref, sem, *, nq):
    j = pl.program_id(0)

    @pl.when(j == 0)
    def _load_queries():
        cp = pltpu.make_async_copy(fva_hbm, fva_ref, sem)
        cp.start()
        cp.wait()

    mba = mba_ref[...]
    for qc in range(nq // _Q_CHUNK):
        sl = pl.ds(qc * _Q_CHUNK, _Q_CHUNK)
        fvc = fva_ref[sl, :]
        s = jax.lax.dot_general(mba, fvc, (((1,), (1,)), ((), ())),
                                preferred_element_type=jnp.float32
                                ).astype(jnp.bfloat16)
        m = jnp.min(s, axis=0, keepdims=True).astype(jnp.float32)
        m = m + (_MB_SHIFT + _FV_SHIFT - 288.0)
        prev = jnp.where(j == 0, jnp.inf, out_ref[:, sl])
        out_ref[:, sl] = jnp.minimum(prev, m)


def _select_kernel(raw_ref, fva_ref, ps_ref, sel_ref, pos_ref, *, hw, nq):
    raw = raw_ref[...]
    ps_ref[...] = jnp.sqrt(raw)
    col = jax.lax.broadcasted_iota(jnp.int32, raw.shape, 1)
    b = nq // hw
    pos_list = []
    taken = jnp.zeros(raw.shape, jnp.bool_)
    for bi in range(b):
        seg = jnp.logical_and(col >= bi * hw, col < (bi + 1) * hw)
        for _ in range(_NCAND):
            cand = jnp.logical_and(seg, jnp.logical_not(taken))
            mx = jnp.max(jnp.where(cand, raw, -jnp.inf))
            hit = jnp.logical_and(cand, raw == mx)
            p = jnp.min(jnp.where(hit, col, nq))
            taken = jnp.logical_or(taken, col == p)
            pos_list.append(jnp.full((1, 1), 0, jnp.int32) + p)
    pos = jnp.concatenate(pos_list, axis=0)
    pos_ref[...] = pos
    qcol = jax.lax.broadcasted_iota(jnp.int32, (b * _NCAND, nq), 1)
    onehot = (qcol == pos).astype(jnp.float32)
    sel_ref[...] = jax.lax.dot_general(
        onehot, fva_ref[...], (((1,), (0,)), ((), ())),
        preferred_element_type=jnp.float32)


def _sel_dist_kernel(sel_ref, mba_ref, d_ref):
    d_ref[...] = jax.lax.dot_general(
        sel_ref[...], mba_ref[...], (((1,), (1,)), ((), ())),
        preferred_element_type=jnp.float32)


def _top9_kernel(d_ref, pos_ref, img_ref, *, nq):
    d = d_ref[...]
    pos = pos_ref[...]
    nc = d.shape[0]
    b = nc // _NCAND
    rmin = jnp.min(d, axis=1, keepdims=True)
    row = jax.lax.broadcasted_iota(jnp.int32, (nc, 1), 0)
    img = row // _NCAND
    best_rows = []
    for bi in range(b):
        mask = img == bi
        mx = jnp.max(jnp.where(mask, rmin, -jnp.inf))
        hit = jnp.logical_and(mask, rmin == mx)
        pb = jnp.min(jnp.where(hit, pos, nq))
        rb = jnp.min(jnp.where(jnp.logical_and(hit, pos == pb), row, nc))
        best_rows.append(jnp.full((1, 1), 0, jnp.int32) + rb)
    best = jnp.concatenate(best_rows, axis=0)
    ccol = jax.lax.broadcasted_iota(jnp.int32, (b, nc), 1)
    oh = (ccol == best).astype(jnp.float32)
    dd = jax.lax.dot_general(oh, d, (((1,), (0,)), ((), ())),
                             preferred_element_type=jnp.float32)
    col = jax.lax.broadcasted_iota(jnp.int32, dd.shape, 1)
    lane = jax.lax.broadcasted_iota(jnp.int32, (b, 16), 1)
    top = jnp.zeros((b, 16), jnp.float32)
    for k in range(_B_NEIGH):
        m = jnp.min(dd, axis=1, keepdims=True)
        p = jnp.min(jnp.where(dd == m, col, dd.shape[1]), axis=1,
                    keepdims=True)
        top = jnp.where(lane == k, m, top)
        dd = jnp.where(col == p, jnp.inf, dd)
    sd = jnp.sqrt(top)
    valid = lane < _B_NEIGH
    mxv = jnp.max(jnp.where(valid, sd, -jnp.inf), axis=1, keepdims=True)
    e = jnp.where(valid, jnp.exp(sd - mxv), 0.0)
    p0 = e[:, 0:1] / jnp.sum(e, axis=1, keepdims=True)
    img_ref[...] = sd[:, 0:1] * (1.0 - p0)


def kernel(feature_batch, mb):
    b, h, w, c = feature_batch.shape
    nq = b * h * w
    m = mb.shape[0]
    nc = b * _NCAND
    fv = jnp.reshape(feature_batch, (nq, c))

    mbaf, mbab = pl.pallas_call(
        _mb_aug_kernel,
        grid=(16,),
        in_specs=[pl.BlockSpec((m // 16, c), lambda j: (j, 0))],
        out_specs=[pl.BlockSpec((m // 16, _K_AUG), lambda j: (j, 0)),
                   pl.BlockSpec((m // 16, _K_AUG), lambda j: (j, 0))],
        out_shape=[jax.ShapeDtypeStruct((m, _K_AUG), jnp.float32),
                   jax.ShapeDtypeStruct((m, _K_AUG), jnp.bfloat16)],
    )(mb)
    fvaf, fvab = pl.pallas_call(
        _fv_aug_kernel,
        in_specs=[pl.BlockSpec((nq, c), lambda: (0, 0))],
        out_specs=[pl.BlockSpec((nq, _K_AUG), lambda: (0, 0)),
                   pl.BlockSpec((nq, _K_AUG), lambda: (0, 0))],
        out_shape=[jax.ShapeDtypeStruct((nq, _K_AUG), jnp.float32),
                   jax.ShapeDtypeStruct((nq, _K_AUG), jnp.bfloat16)],
    )(fv)

    n1 = m // _MB_BLK1
    raw = pl.pallas_call(
        functools.partial(_pixel_min_kernel, nq=nq),
        grid=(n1,),
        in_specs=[pl.BlockSpec(memory_space=pl.ANY),
                  pl.BlockSpec((_MB_BLK1, _K_AUG), lambda j: (j, 0))],
        out_specs=pl.BlockSpec((1, nq), lambda j: (0, 0)),
        out_shape=jax.ShapeDtypeStruct((1, nq), jnp.float32),
        scratch_shapes=[pltpu.VMEM((nq, _K_AUG), jnp.bfloat16),
                        pltpu.SemaphoreType.DMA],
        compiler_params=pltpu.CompilerParams(
            dimension_semantics=("arbitrary",)),
    )(fvab, mbab)

    ps, sel, pos = pl.pallas_call(
        functools.partial(_select_kernel, hw=h * w, nq=nq),
        in_specs=[pl.BlockSpec((1, nq), lambda: (0, 0)),
                  pl.BlockSpec((nq, _K_AUG), lambda: (0, 0))],
        out_specs=[pl.BlockSpec((1, nq), lambda: (0, 0)),
                   pl.BlockSpec((nc, _K_AUG), lambda: (0, 0)),
                   pl.BlockSpec((nc, 1), lambda: (0, 0))],
        out_shape=[jax.ShapeDtypeStruct((1, nq), jnp.float32),
                   jax.ShapeDtypeStruct((nc, _K_AUG), jnp.float32),
                   jax.ShapeDtypeStruct((nc, 1), jnp.int32)],
    )(raw, fvaf)

    n2 = m // _MB_BLK2
    d = pl.pallas_call(
        _sel_dist_kernel,
        grid=(n2,),
        in_specs=[pl.BlockSpec((nc, _K_AUG), lambda j: (0, 0)),
                  pl.BlockSpec((_MB_BLK2, _K_AUG), lambda j: (j, 0))],
        out_specs=pl.BlockSpec((nc, _MB_BLK2), lambda j: (0, j)),
        out_shape=jax.ShapeDtypeStruct((nc, m), jnp.float32),
        compiler_params=pltpu.CompilerParams(
            dimension_semantics=("arbitrary",)),
    )(sel, mbaf)

    img = pl.pallas_call(
        functools.partial(_top9_kernel, nq=nq),
        in_specs=[pl.BlockSpec((nc, m), lambda: (0, 0)),
                  pl.BlockSpec((nc, 1), lambda: (0, 0))],
        out_specs=pl.BlockSpec((b, 1), lambda: (0, 0)),
        out_shape=jax.ShapeDtypeStruct((b, 1), jnp.float32),
    )(d, pos)

    pixel_scores = jnp.reshape(ps, (b, 1, h, w))
    image_scores = img[:, 0]
    return (pixel_scores, image_scores)


# speedup vs baseline: 1.1015x; 1.1015x over previous
import functools

import jax
import jax.numpy as jnp
from jax.experimental import pallas as pl
from jax.experimental.pallas import tpu as pltpu

_B_NEIGH = 9
_K_AUG = 136
_MB_BLK1 = 2048
_Q_CHUNK = 448
_MB_BLK2 = 4096


def _mb_aug_kernel(mb_ref, out_ref):
    mbb = mb_ref[...]
    c = mbb.shape[1]
    mbn = jnp.sum(mbb * mbb, axis=1, keepdims=True)
    one = jnp.ones((mbb.shape[0], 1), jnp.float32)
    z = jnp.zeros((mbb.shape[0], _K_AUG - c - 2), jnp.float32)
    out_ref[...] = jnp.concatenate([mbb * -2.0, mbn, one, z], axis=1)


def _fv_aug_kernel(fv_ref, out_ref):
    fvv = fv_ref[...]
    c = fvv.shape[1]
    qn = jnp.sum(fvv * fvv, axis=1, keepdims=True)
    one = jnp.ones((fvv.shape[0], 1), jnp.float32)
    z = jnp.zeros((fvv.shape[0], _K_AUG - c - 2), jnp.float32)
    out_ref[...] = jnp.concatenate([fvv, one, qn, z], axis=1)


def _pixel_min_kernel(fva_ref, mba_ref, out_ref, *, nq):
    j = pl.program_id(0)
    mba = mba_ref[...]
    for qc in range(nq // _Q_CHUNK):
        sl = pl.ds(qc * _Q_CHUNK, _Q_CHUNK)
        fvc = fva_ref[sl, :]
        s = jax.lax.dot_general(mba, fvc, (((1,), (1,)), ((), ())),
                                preferred_element_type=jnp.float32)
        m = jnp.min(s, axis=0, keepdims=True)
        prev = jnp.where(j == 0, jnp.inf, out_ref[:, sl])
        out_ref[:, sl] = jnp.minimum(prev, m)


def _image_score_kernel(raw_ref, fva_ref, mba_ref, img_ref, ps_ref,
                        sel_ref, d_ref, *, n_blocks, hw, nq, blk):
    j = pl.program_id(0)

    @pl.when(j == 0)
    def _select():
        ps = jnp.sqrt(raw_ref[...])
        ps_ref[...] = ps
        col = jax.lax.broadcasted_iota(jnp.int32, ps.shape, 1)
        b = nq // hw
        pos_list = []
        for bi in range(b):
            seg = jnp.logical_and(col >= bi * hw, col < (bi + 1) * hw)
            mx = jnp.max(jnp.where(seg, ps, -jnp.inf))
            p = jnp.min(jnp.where(jnp.logical_and(seg, ps == mx), col, nq))
            pos_list.append(jnp.full((1, 1), 0, jnp.int32) + p)
        pos = jnp.concatenate(pos_list, axis=0)
        qcol = jax.lax.broadcasted_iota(jnp.int32, (b, nq), 1)
        onehot = (qcol == pos).astype(jnp.float32)
        sel_ref[...] = jax.lax.dot_general(
            onehot, fva_ref[...], (((1,), (0,)), ((), ())),
            preferred_element_type=jnp.float32)

    sel = sel_ref[...]
    s = jax.lax.dot_general(sel, mba_ref[...], (((1,), (1,)), ((), ())),
                            preferred_element_type=jnp.float32)
    d_ref[:, pl.ds(j * blk, blk)] = s

    @pl.when(j == n_blocks - 1)
    def _fin():
        _finalize(d_ref, img_ref)


def _finalize(d_ref, img_ref):
    d = d_ref[...]
    bsz = d.shape[0]
    col = jax.lax.broadcasted_iota(jnp.int32, d.shape, 1)
    lane = jax.lax.broadcasted_iota(jnp.int32, (bsz, 16), 1)
    top = jnp.zeros((bsz, 16), jnp.float32)
    for k in range(_B_NEIGH):
        m = jnp.min(d, axis=1, keepdims=True)
        p = jnp.min(jnp.where(d == m, col, d.shape[1]), axis=1,
                    keepdims=True)
        top = jnp.where(lane == k, m, top)
        d = jnp.where(col == p, jnp.inf, d)
    sd = jnp.sqrt(top)
    valid = lane < _B_NEIGH
    mxv = jnp.max(jnp.where(valid, sd, -jnp.inf), axis=1, keepdims=True)
    e = jnp.where(valid, jnp.exp(sd - mxv), 0.0)
    p0 = e[:, 0:1] / jnp.sum(e, axis=1, keepdims=True)
    img_ref[...] = sd[:, 0:1] * (1.0 - p0)


def kernel(feature_batch, mb):
    b, h, w, c = feature_batch.shape
    nq = b * h * w
    m = mb.shape[0]
    fv = jnp.reshape(feature_batch, (nq, c))

    mba = pl.pallas_call(
        _mb_aug_kernel,
        grid=(16,),
        in_specs=[pl.BlockSpec((m // 16, c), lambda j: (j, 0))],
        out_specs=pl.BlockSpec((m // 16, _K_AUG), lambda j: (j, 0)),
        out_shape=jax.ShapeDtypeStruct((m, _K_AUG), jnp.float32),
    )(mb)
    fva = pl.pallas_call(
        _fv_aug_kernel,
        in_specs=[pl.BlockSpec((nq, c), lambda: (0, 0))],
        out_specs=pl.BlockSpec((nq, _K_AUG), lambda: (0, 0)),
        out_shape=jax.ShapeDtypeStruct((nq, _K_AUG), jnp.float32),
    )(fv)

    n1 = m // _MB_BLK1
    raw = pl.pallas_call(
        functools.partial(_pixel_min_kernel, nq=nq),
        grid=(n1,),
        in_specs=[pl.BlockSpec((nq, _K_AUG), lambda j: (0, 0)),
                  pl.BlockSpec((_MB_BLK1, _K_AUG), lambda j: (j, 0))],
        out_specs=pl.BlockSpec((1, nq), lambda j: (0, 0)),
        out_shape=jax.ShapeDtypeStruct((1, nq), jnp.float32),
        compiler_params=pltpu.CompilerParams(
            dimension_semantics=("arbitrary",)),
    )(fva, mba)

    n2 = m // _MB_BLK2
    img, ps = pl.pallas_call(
        functools.partial(_image_score_kernel, n_blocks=n2, hw=h * w,
                          nq=nq, blk=_MB_BLK2),
        grid=(n2,),
        in_specs=[pl.BlockSpec((1, nq), lambda j: (0, 0)),
                  pl.BlockSpec((nq, _K_AUG), lambda j: (0, 0)),
                  pl.BlockSpec((_MB_BLK2, _K_AUG), lambda j: (j, 0))],
        out_specs=[pl.BlockSpec((b, 1), lambda j: (0, 0)),
                   pl.BlockSpec((1, nq), lambda j: (0, 0))],
        out_shape=[jax.ShapeDtypeStruct((b, 1), jnp.float32),
                   jax.ShapeDtypeStruct((1, nq), jnp.float32)],
        scratch_shapes=[pltpu.VMEM((b, _K_AUG), jnp.float32),
                        pltpu.VMEM((b, m), jnp.float32)],
        compiler_params=pltpu.CompilerParams(
            dimension_semantics=("arbitrary",)),
    )(raw, fva, mba)

    pixel_scores = jnp.reshape(ps, (b, 1, h, w))
    image_scores = img[:, 0]
    return (pixel_scores, image_scores)


# speedup vs baseline: 1.1457x; 1.0401x over previous
import functools

import jax
import jax.numpy as jnp
from jax.experimental import pallas as pl
from jax.experimental.pallas import tpu as pltpu

_B_NEIGH = 9
_K_AUG = 136
_MB_BLK1 = 2048
_Q_CHUNK = 448
_MB_BLK2 = 8192


def _mb_aug_kernel(mb_ref, out_ref):
    mbb = mb_ref[...]
    c = mbb.shape[1]
    mbn = jnp.sum(mbb * mbb, axis=1, keepdims=True)
    one = jnp.ones((mbb.shape[0], 1), jnp.float32)
    z = jnp.zeros((mbb.shape[0], _K_AUG - c - 2), jnp.float32)
    out_ref[...] = jnp.concatenate([mbb * -2.0, mbn, one, z], axis=1)


def _fv_aug_kernel(fv_ref, out_ref):
    fvv = fv_ref[...]
    c = fvv.shape[1]
    qn = jnp.sum(fvv * fvv, axis=1, keepdims=True)
    one = jnp.ones((fvv.shape[0], 1), jnp.float32)
    z = jnp.zeros((fvv.shape[0], _K_AUG - c - 2), jnp.float32)
    out_ref[...] = jnp.concatenate([fvv, one, qn, z], axis=1)


def _pixel_min_kernel(fva_ref, mba_ref, out_ref, *, nq):
    j = pl.program_id(0)
    mba = mba_ref[...]
    for qc in range(nq // _Q_CHUNK):
        sl = pl.ds(qc * _Q_CHUNK, _Q_CHUNK)
        fvc = fva_ref[sl, :]
        s = jax.lax.dot_general(mba, fvc, (((1,), (1,)), ((), ())),
                                preferred_element_type=jnp.float32)
        m = jnp.min(s, axis=0, keepdims=True)
        prev = jnp.where(j == 0, jnp.inf, out_ref[:, sl])
        out_ref[:, sl] = jnp.minimum(prev, m)


def _image_score_kernel(raw_ref, fva_ref, mba_ref, img_ref, ps_ref,
                        sel_ref, d_ref, *, n_blocks, hw, nq, blk):
    j = pl.program_id(0)

    @pl.when(j == 0)
    def _select():
        ps = jnp.sqrt(raw_ref[...])
        ps_ref[...] = ps
        col = jax.lax.broadcasted_iota(jnp.int32, ps.shape, 1)
        b = nq // hw
        pos_list = []
        for bi in range(b):
            seg = jnp.logical_and(col >= bi * hw, col < (bi + 1) * hw)
            mx = jnp.max(jnp.where(seg, ps, -jnp.inf))
            p = jnp.min(jnp.where(jnp.logical_and(seg, ps == mx), col, nq))
            pos_list.append(jnp.full((1, 1), 0, jnp.int32) + p)
        pos = jnp.concatenate(pos_list, axis=0)
        qcol = jax.lax.broadcasted_iota(jnp.int32, (b, nq), 1)
        onehot = (qcol == pos).astype(jnp.float32)
        sel_ref[...] = jax.lax.dot_general(
            onehot, fva_ref[...], (((1,), (0,)), ((), ())),
            preferred_element_type=jnp.float32)

    sel = sel_ref[...]
    s = jax.lax.dot_general(sel, mba_ref[...], (((1,), (1,)), ((), ())),
                            preferred_element_type=jnp.float32)
    d_ref[:, pl.ds(j * blk, blk)] = s

    @pl.when(j == n_blocks - 1)
    def _fin():
        _finalize(d_ref, img_ref)


def _finalize(d_ref, img_ref):
    d = d_ref[...]
    bsz = d.shape[0]
    col = jax.lax.broadcasted_iota(jnp.int32, d.shape, 1)
    lane = jax.lax.broadcasted_iota(jnp.int32, (bsz, 16), 1)
    top = jnp.zeros((bsz, 16), jnp.float32)
    for k in range(_B_NEIGH):
        m = jnp.min(d, axis=1, keepdims=True)
        p = jnp.min(jnp.where(d == m, col, d.shape[1]), axis=1,
                    keepdims=True)
        top = jnp.where(lane == k, m, top)
        d = jnp.where(col == p, jnp.inf, d)
    sd = jnp.sqrt(top)
    valid = lane < _B_NEIGH
    mxv = jnp.max(jnp.where(valid, sd, -jnp.inf), axis=1, keepdims=True)
    e = jnp.where(valid, jnp.exp(sd - mxv), 0.0)
    p0 = e[:, 0:1] / jnp.sum(e, axis=1, keepdims=True)
    img_ref[...] = sd[:, 0:1] * (1.0 - p0)


def kernel(feature_batch, mb):
    b, h, w, c = feature_batch.shape
    nq = b * h * w
    m = mb.shape[0]
    fv = jnp.reshape(feature_batch, (nq, c))

    mba = pl.pallas_call(
        _mb_aug_kernel,
        grid=(8,),
        in_specs=[pl.BlockSpec((m // 8, c), lambda j: (j, 0))],
        out_specs=pl.BlockSpec((m // 8, _K_AUG), lambda j: (j, 0)),
        out_shape=jax.ShapeDtypeStruct((m, _K_AUG), jnp.float32),
    )(mb)
    fva = pl.pallas_call(
        _fv_aug_kernel,
        in_specs=[pl.BlockSpec((nq, c), lambda: (0, 0))],
        out_specs=pl.BlockSpec((nq, _K_AUG), lambda: (0, 0)),
        out_shape=jax.ShapeDtypeStruct((nq, _K_AUG), jnp.float32),
    )(fv)

    n1 = m // _MB_BLK1
    raw = pl.pallas_call(
        functools.partial(_pixel_min_kernel, nq=nq),
        grid=(n1,),
        in_specs=[pl.BlockSpec((nq, _K_AUG), lambda j: (0, 0)),
                  pl.BlockSpec((_MB_BLK1, _K_AUG), lambda j: (j, 0))],
        out_specs=pl.BlockSpec((1, nq), lambda j: (0, 0)),
        out_shape=jax.ShapeDtypeStruct((1, nq), jnp.float32),
        compiler_params=pltpu.CompilerParams(
            dimension_semantics=("arbitrary",)),
    )(fva, mba)

    n2 = m // _MB_BLK2
    img, ps = pl.pallas_call(
        functools.partial(_image_score_kernel, n_blocks=n2, hw=h * w,
                          nq=nq, blk=_MB_BLK2),
        grid=(n2,),
        in_specs=[pl.BlockSpec((1, nq), lambda j: (0, 0)),
                  pl.BlockSpec((nq, _K_AUG), lambda j: (0, 0)),
                  pl.BlockSpec((_MB_BLK2, _K_AUG), lambda j: (j, 0))],
        out_specs=[pl.BlockSpec((b, 1), lambda j: (0, 0)),
                   pl.BlockSpec((1, nq), lambda j: (0, 0))],
        out_shape=[jax.ShapeDtypeStruct((b, 1), jnp.float32),
                   jax.ShapeDtypeStruct((1, nq), jnp.float32)],
        scratch_shapes=[pltpu.VMEM((b, _K_AUG), jnp.float32),
                        pltpu.VMEM((b, m), jnp.float32)],
        compiler_params=pltpu.CompilerParams(
            dimension_semantics=("arbitrary",)),
    )(raw, fva, mba)

    pixel_scores = jnp.reshape(ps, (b, 1, h, w))
    image_scores = img[:, 0]
    return (pixel_scores, image_scores)
